# baseline (speedup 1.0000x reference)
import jax
import jax.numpy as jnp
from jax.experimental import pallas as pl
from jax.experimental.pallas import tpu as pltpu

POP = 4096
KEY_DIM = 1024
HIDDEN = 64
N_SPLIT = 4
GRID = 4
SHARD = POP // N_SPLIT
BLOCK = SHARD // GRID


def _brkga_fitness_kernel(*refs):
    x_refs = refs[:N_SPLIT]
    w_ref = refs[N_SPLIT]
    out_refs = refs[N_SPLIT + 1:]
    w = w_ref[...]
    for x_ref, out_ref in zip(x_refs, out_refs):
        h = jnp.dot(x_ref[...], w, preferred_element_type=jnp.float32)
        out_ref[...] = jnp.sum(jnp.maximum(h, 0.0), axis=1, keepdims=True)


def kernel(keys_pop, W):
    shards = [
        jax.lax.slice_in_dim(keys_pop, s * SHARD, (s + 1) * SHARD, axis=0)
        for s in range(N_SPLIT)
    ]
    outs = pl.pallas_call(
        _brkga_fitness_kernel,
        grid=(GRID,),
        in_specs=[pl.BlockSpec((BLOCK, KEY_DIM), lambda i: (i, 0))
                  for _ in range(N_SPLIT)]
        + [pl.BlockSpec((KEY_DIM, HIDDEN), lambda i: (0, 0))],
        out_specs=[pl.BlockSpec((BLOCK, 1), lambda i: (i, 0))
                   for _ in range(N_SPLIT)],
        out_shape=[jax.ShapeDtypeStruct((SHARD, 1), jnp.float32)
                   for _ in range(N_SPLIT)],
        compiler_params=pltpu.CompilerParams(
            dimension_semantics=("parallel",),
        ),
    )(*shards, W)
    return jnp.concatenate(outs, axis=0).reshape(POP)


# speedup vs baseline: 1.6995x; 1.6995x over previous
import jax
import jax.numpy as jnp
from jax.experimental import pallas as pl
from jax.experimental.pallas import tpu as pltpu

POP = 4096
KEY_DIM = 1024
HIDDEN = 64
N_SPLIT = 4
GRID = 4
SHARD = POP // N_SPLIT
BLOCK = SHARD // GRID


def _brkga_fitness_kernel(*refs):
    x_refs = refs[:N_SPLIT]
    w_ref = refs[N_SPLIT]
    out_refs = refs[N_SPLIT + 1:]
    w = w_ref[...]
    for x_ref, out_ref in zip(x_refs, out_refs):
        h = jnp.dot(x_ref[...], w, preferred_element_type=jnp.float32)
        out_ref[...] = jnp.sum(jnp.maximum(h, 0.0), axis=1, keepdims=True)


def kernel(keys_pop, W):
    outs = pl.pallas_call(
        _brkga_fitness_kernel,
        grid=(GRID,),
        in_specs=[pl.BlockSpec((BLOCK, KEY_DIM),
                               lambda i, s=s: (s * GRID + i, 0))
                  for s in range(N_SPLIT)]
        + [pl.BlockSpec((KEY_DIM, HIDDEN), lambda i: (0, 0))],
        out_specs=[pl.BlockSpec((BLOCK, 1), lambda i: (i, 0))
                   for _ in range(N_SPLIT)],
        out_shape=[jax.ShapeDtypeStruct((SHARD, 1), jnp.float32)
                   for _ in range(N_SPLIT)],
        compiler_params=pltpu.CompilerParams(
            dimension_semantics=("parallel",),
        ),
    )(*([keys_pop] * N_SPLIT), W)
    return jnp.concatenate(outs, axis=0).reshape(POP)


# speedup vs baseline: 2.0321x; 1.1957x over previous
import jax
import jax.numpy as jnp
from jax.experimental import pallas as pl
from jax.experimental.pallas import tpu as pltpu

POP = 4096
KEY_DIM = 1024
HIDDEN = 64
BLOCK = 1024


def _brkga_fitness_kernel(x_ref, w_ref, out_ref):
    h = jnp.dot(x_ref[...], w_ref[...], preferred_element_type=jnp.float32)
    out_ref[...] = jnp.sum(jnp.maximum(h, 0.0), axis=1, keepdims=True)


def kernel(keys_pop, W):
    grid = (POP // BLOCK,)
    out = pl.pallas_call(
        _brkga_fitness_kernel,
        grid=grid,
        in_specs=[
            pl.BlockSpec((BLOCK, KEY_DIM), lambda i: (i, 0)),
            pl.BlockSpec((KEY_DIM, HIDDEN), lambda i: (0, 0)),
        ],
        out_specs=pl.BlockSpec((BLOCK, 1), lambda i: (i, 0)),
        out_shape=jax.ShapeDtypeStruct((POP, 1), jnp.float32),
        compiler_params=pltpu.CompilerParams(
            dimension_semantics=("arbitrary",),
        ),
    )(keys_pop, W)
    return out.reshape(POP)
